# baseline (speedup 1.0000x reference)
import functools

import jax
import jax.numpy as jnp
from jax import lax
from jax.experimental import pallas as pl
from jax.experimental.pallas import tpu as pltpu
from jax.experimental.pallas import tpu_sc as plsc

MAX_REL = 128
DIM = 64
ROWS = 2 * MAX_REL + 1
SEQ = 2048
NC = 2
NS = 16
NW = NC * NS
BJ = 4352
JC = BJ // NW
QCH = 256
WW = 384
SW = 4096


def _sc_build_bt(position_embeddings):
    mesh = plsc.VectorSubcoreMesh(core_axis_name="c", subcore_axis_name="s")

    @functools.partial(
        pl.kernel,
        mesh=mesh,
        out_type=jax.ShapeDtypeStruct((DIM, BJ), jnp.float32),
        scratch_types=[
            pltpu.VMEM((ROWS, DIM), jnp.float32),
            pltpu.VMEM((DIM, JC), jnp.float32),
        ],
        compiler_params=pltpu.CompilerParams(
            use_tc_tiling_on_sc=False, needs_layout_passes=False),
    )
    def run(table_hbm, out_hbm, table_v, bt_v):
        wid = lax.axis_index("s") * NC + lax.axis_index("c")
        j0 = wid * JC

        pltpu.sync_copy(table_hbm, table_v)

        lanes = lax.iota(jnp.int32, 16)

        def build(j, carry):
            idx = jnp.clip(j0 + j - 1919, 0, ROWS - 1)
            jvec = jnp.full((16,), 0, jnp.int32) + j
            for c in range(DIM // 16):
                v = table_v[idx, pl.ds(16 * c, 16)]
                plsc.store_scatter(bt_v, [16 * c + lanes, jvec], v)
            return carry

        lax.fori_loop(0, JC, build, 0, unroll=False)

        pltpu.sync_copy(bt_v, out_hbm.at[:, pl.ds(j0, JC)])

    return run(position_embeddings)


def _tc_materialize(b_t):
    def body(bt_ref, out_ref, s_ref):
        r = pl.program_id(0)
        m = pl.program_id(1)
        ph = (SEQ - 1 - r) % 128

        @pl.when(m == 0)
        def _stage():
            for ch in range(SW // QCH):
                w = bt_ref[:, pl.ds(QCH * ch, WW)]
                s_ref[:, pl.ds(QCH * ch, QCH)] = pltpu.roll(w, -ph, 1)[:, :QCH]

        off = 128 * (SEQ // 128 - 1 - m)
        out_ref[0] = s_ref[:, pl.ds(off, SEQ)]

    return pl.pallas_call(
        body,
        grid=(128, SEQ // 128),
        in_specs=[pl.BlockSpec((DIM, BJ), lambda r, m: (0, 0))],
        out_specs=pl.BlockSpec(
            (1, DIM, SEQ), lambda r, m: (128 * m + r, 0, 0)),
        out_shape=jax.ShapeDtypeStruct((SEQ, DIM, SEQ), jnp.float32),
        scratch_shapes=[pltpu.VMEM((DIM, SW), jnp.float32)],
    )(b_t)


def kernel(query_length, key_length, position_embeddings):
    del query_length, key_length
    p = _tc_materialize(_sc_build_bt(position_embeddings))
    return p.transpose(0, 2, 1)


# speedup vs baseline: 1.7907x; 1.7907x over previous
import functools

import jax
import jax.numpy as jnp
from jax import lax
from jax.experimental import pallas as pl
from jax.experimental.pallas import tpu as pltpu
from jax.experimental.pallas import tpu_sc as plsc

MAX_REL = 128
DIM = 64
ROWS = 2 * MAX_REL + 1
SEQ = 2048
NC = 2
NS = 16
NW = NC * NS
BJ = 4352
JC = BJ // NW
BK = 16
QCH = 256
WW = 512


def _sc_build_bt(position_embeddings):
    mesh = plsc.VectorSubcoreMesh(core_axis_name="c", subcore_axis_name="s")

    @functools.partial(
        pl.kernel,
        mesh=mesh,
        out_type=jax.ShapeDtypeStruct((DIM, BJ), jnp.float32),
        scratch_types=[
            pltpu.VMEM((ROWS, DIM), jnp.float32),
            pltpu.VMEM((DIM, JC), jnp.float32),
        ],
        compiler_params=pltpu.CompilerParams(
            use_tc_tiling_on_sc=False, needs_layout_passes=False),
    )
    def run(table_hbm, out_hbm, table_v, bt_v):
        wid = lax.axis_index("s") * NC + lax.axis_index("c")
        j0 = wid * JC

        pltpu.sync_copy(table_hbm, table_v)

        lanes = lax.iota(jnp.int32, 16)

        def build(j, carry):
            idx = jnp.clip(j0 + j - 1919, 0, ROWS - 1)
            jvec = jnp.full((16,), 0, jnp.int32) + j
            for c in range(DIM // 16):
                v = table_v[idx, pl.ds(16 * c, 16)]
                plsc.store_scatter(bt_v, [16 * c + lanes, jvec], v)
            return carry

        lax.fori_loop(0, JC, build, 0, unroll=False)

        pltpu.sync_copy(bt_v, out_hbm.at[:, pl.ds(j0, JC)])

    return run(position_embeddings)


def _tc_materialize(b_t):
    def body(bt_ref, out_ref):
        ib = pl.program_id(0)
        base_last = SEQ - 1 - (ib * BK + (BK - 1))
        aligned = pl.multiple_of((base_last // 128) * 128, 128)
        ph_last = base_last - aligned
        for ch in range(SEQ // QCH):
            w = bt_ref[:, pl.ds(aligned + QCH * ch, WW)]
            rolled = pltpu.roll(w, -ph_last, 1)
            for i in range(BK):
                off = BK - 1 - i
                out_ref[i, :, pl.ds(QCH * ch, QCH)] = (
                    rolled[:, off:off + QCH])

    return pl.pallas_call(
        body,
        grid=(SEQ // BK,),
        in_specs=[pl.BlockSpec((DIM, BJ), lambda ib: (0, 0))],
        out_specs=pl.BlockSpec((BK, DIM, SEQ), lambda ib: (ib, 0, 0)),
        out_shape=jax.ShapeDtypeStruct((SEQ, DIM, SEQ), jnp.float32),
    )(b_t)


def kernel(query_length, key_length, position_embeddings):
    del query_length, key_length
    p = _tc_materialize(_sc_build_bt(position_embeddings))
    return p.transpose(0, 2, 1)


# speedup vs baseline: 1.9195x; 1.0719x over previous
import functools

import jax
import jax.numpy as jnp
from jax import lax
from jax.experimental import pallas as pl
from jax.experimental.pallas import tpu as pltpu
from jax.experimental.pallas import tpu_sc as plsc

MAX_REL = 128
DIM = 64
ROWS = 2 * MAX_REL + 1
SEQ = 2048
NC = 2
NS = 16
NW = NC * NS
BJ = 4352
JC = BJ // NW
BK = 32
QCH = 256
WW = 512


def _sc_build_bt(position_embeddings):
    mesh = plsc.VectorSubcoreMesh(core_axis_name="c", subcore_axis_name="s")

    @functools.partial(
        pl.kernel,
        mesh=mesh,
        out_type=jax.ShapeDtypeStruct((DIM, BJ), jnp.float32),
        scratch_types=[
            pltpu.VMEM((ROWS, DIM), jnp.float32),
            pltpu.VMEM((DIM, JC), jnp.float32),
        ],
        compiler_params=pltpu.CompilerParams(
            use_tc_tiling_on_sc=False, needs_layout_passes=False),
    )
    def run(table_hbm, out_hbm, table_v, bt_v):
        wid = lax.axis_index("s") * NC + lax.axis_index("c")
        j0 = wid * JC

        pltpu.sync_copy(table_hbm, table_v)

        lanes = lax.iota(jnp.int32, 16)

        def build(j, carry):
            idx = jnp.clip(j0 + j - 1919, 0, ROWS - 1)
            jvec = jnp.full((16,), 0, jnp.int32) + j
            for c in range(DIM // 16):
                v = table_v[idx, pl.ds(16 * c, 16)]
                plsc.store_scatter(bt_v, [16 * c + lanes, jvec], v)
            return carry

        lax.fori_loop(0, JC, build, 0, unroll=False)

        pltpu.sync_copy(bt_v, out_hbm.at[:, pl.ds(j0, JC)])

    return run(position_embeddings)


def _tc_materialize(b_t):
    def body(bt_ref, out_ref):
        ib = pl.program_id(0)
        base_last = SEQ - 1 - (ib * BK + (BK - 1))
        aligned = pl.multiple_of((base_last // 128) * 128, 128)
        ph_last = base_last - aligned
        for ch in range(SEQ // QCH):
            w = bt_ref[:, pl.ds(aligned + QCH * ch, WW)]
            rolled = pltpu.roll(w, -ph_last, 1)
            for i in range(BK):
                off = BK - 1 - i
                out_ref[i, :, pl.ds(QCH * ch, QCH)] = (
                    rolled[:, off:off + QCH])

    return pl.pallas_call(
        body,
        grid=(SEQ // BK,),
        in_specs=[pl.BlockSpec((DIM, BJ), lambda ib: (0, 0))],
        out_specs=pl.BlockSpec((BK, DIM, SEQ), lambda ib: (ib, 0, 0)),
        out_shape=jax.ShapeDtypeStruct((SEQ, DIM, SEQ), jnp.float32),
    )(b_t)


def kernel(query_length, key_length, position_embeddings):
    del query_length, key_length
    p = _tc_materialize(_sc_build_bt(position_embeddings))
    return p.transpose(0, 2, 1)
